# baseline (speedup 1.0000x reference)
import functools

import jax
import jax.numpy as jnp
from jax import lax
from jax.experimental import pallas as pl
from jax.experimental.pallas import tpu as pltpu
from jax.experimental.pallas import tpu_sc as plsc

D = 32
IPG = 1024
CH_ROWS = 1
CH = IPG * CH_ROWS


@functools.cache
def _make_emb(n_total):
    info = plsc.get_sparse_core_info()
    NC, NS = info.num_cores, info.num_subcores
    NW = NC * NS
    assert n_total % (NW * CH) == 0, (n_total, NW, CH)
    n_per_w = n_total // NW
    n_chunks = n_per_w // CH
    assert n_chunks % 2 == 0 and n_chunks >= 4
    rows_per_w = n_per_w // IPG

    mesh = plsc.VectorSubcoreMesh(core_axis_name="c", subcore_axis_name="s")

    @functools.partial(
        pl.kernel,
        mesh=mesh,
        out_type=jax.ShapeDtypeStruct((n_total, D), jnp.float32),
        scratch_types=[
            pltpu.VMEM((CH_ROWS, IPG), jnp.int32),
            pltpu.VMEM((CH_ROWS, IPG), jnp.int32),
            pltpu.VMEM((CH, D), jnp.float32),
            pltpu.VMEM((CH, D), jnp.float32),
            pltpu.SemaphoreType.DMA,
            pltpu.SemaphoreType.DMA,
            pltpu.SemaphoreType.DMA,
        ],
        compiler_params=pltpu.CompilerParams(use_tc_tiling_on_sc=False),
    )
    def emb(idx_hbm, table_hbm, out_flat, idx0, idx1, rows0, rows1,
            gsem0, gsem1, osem):
        out_hbm = out_flat
        wid = lax.axis_index("s") * NC + lax.axis_index("c")
        row_base = wid * rows_per_w

        idx_b = (idx0, idx1)
        rows_b = (rows0, rows1)
        gsem_b = (gsem0, gsem1)

        def load_idx(c, b):
            pltpu.sync_copy(
                idx_hbm.at[pl.ds(row_base + c * CH_ROWS, CH_ROWS)], idx_b[b])

        def fire_gathers(b):
            for j in range(CH_ROWS):
                pltpu.async_copy(
                    table_hbm.at[idx_b[b].at[j]],
                    rows_b[b].at[pl.ds(j * IPG, IPG)],
                    gsem_b[b],
                )

        def drain_gathers(b):
            pltpu.make_async_copy(
                out_hbm.at[pl.ds(0, CH)], rows_b[b], gsem_b[b]).wait()

        def out_slice(c):
            return out_hbm.at[pl.ds((row_base + c * CH_ROWS) * IPG, CH)]

        def fire_wb(c, b):
            pltpu.async_copy(rows_b[b], out_slice(c), osem)

        def drain_wb(c, b):
            pltpu.make_async_copy(rows_b[b], out_slice(c), osem).wait()

        load_idx(0, 0)
        fire_gathers(0)
        load_idx(1, 1)

        def body(i, carry):
            for x in (0, 1):
                c = i * 2 + x
                y = 1 - x

                @pl.when(c > 0)
                def _():
                    drain_wb(c - 1, y)

                @pl.when(c + 1 < n_chunks)
                def _():
                    fire_gathers(y)

                drain_gathers(x)
                fire_wb(c, x)

                @pl.when(c + 2 < n_chunks)
                def _():
                    load_idx(c + 2, x)
            return carry

        lax.fori_loop(0, n_chunks // 2, body, 0)
        drain_wb(n_chunks - 1, (n_chunks - 1) % 2)

    return emb


def kernel(residue_type, weight):
    b, h = residue_type.shape
    n = b * h
    idx = residue_type.reshape(n // IPG, IPG).astype(jnp.int32)
    out = _make_emb(n)(idx, weight)
    return out.reshape(b, h, weight.shape[1])


# speedup vs baseline: 1.0005x; 1.0005x over previous
import functools

import jax
import jax.numpy as jnp
from jax import lax
from jax.experimental import pallas as pl
from jax.experimental.pallas import tpu as pltpu
from jax.experimental.pallas import tpu_sc as plsc

D = 32
IPG = 512
CH_ROWS = 2
CH = IPG * CH_ROWS


@functools.cache
def _make_emb(n_total):
    info = plsc.get_sparse_core_info()
    NC, NS = info.num_cores, info.num_subcores
    NW = NC * NS
    assert n_total % (NW * CH) == 0, (n_total, NW, CH)
    n_per_w = n_total // NW
    n_chunks = n_per_w // CH
    assert n_chunks % 2 == 0 and n_chunks >= 4
    rows_per_w = n_per_w // IPG

    mesh = plsc.VectorSubcoreMesh(core_axis_name="c", subcore_axis_name="s")

    @functools.partial(
        pl.kernel,
        mesh=mesh,
        out_type=jax.ShapeDtypeStruct((n_total, D), jnp.float32),
        scratch_types=[
            pltpu.VMEM((CH_ROWS, IPG), jnp.int32),
            pltpu.VMEM((CH_ROWS, IPG), jnp.int32),
            pltpu.VMEM((CH, D), jnp.float32),
            pltpu.VMEM((CH, D), jnp.float32),
            pltpu.SemaphoreType.DMA,
            pltpu.SemaphoreType.DMA,
            pltpu.SemaphoreType.DMA,
        ],
        compiler_params=pltpu.CompilerParams(use_tc_tiling_on_sc=False),
    )
    def emb(idx_hbm, table_hbm, out_flat, idx0, idx1, rows0, rows1,
            gsem0, gsem1, osem):
        out_hbm = out_flat
        wid = lax.axis_index("s") * NC + lax.axis_index("c")
        row_base = wid * rows_per_w

        idx_b = (idx0, idx1)
        rows_b = (rows0, rows1)
        gsem_b = (gsem0, gsem1)

        def load_idx(c, b):
            pltpu.sync_copy(
                idx_hbm.at[pl.ds(row_base + c * CH_ROWS, CH_ROWS)], idx_b[b])

        def fire_gathers(b):
            for j in range(CH_ROWS):
                pltpu.async_copy(
                    table_hbm.at[idx_b[b].at[j]],
                    rows_b[b].at[pl.ds(j * IPG, IPG)],
                    gsem_b[b],
                )

        def drain_gathers(b):
            pltpu.make_async_copy(
                out_hbm.at[pl.ds(0, CH)], rows_b[b], gsem_b[b]).wait()

        def out_slice(c):
            return out_hbm.at[pl.ds((row_base + c * CH_ROWS) * IPG, CH)]

        def fire_wb(c, b):
            pltpu.async_copy(rows_b[b], out_slice(c), osem)

        def drain_wb(c, b):
            pltpu.make_async_copy(rows_b[b], out_slice(c), osem).wait()

        load_idx(0, 0)
        fire_gathers(0)
        load_idx(1, 1)

        def body(i, carry):
            for x in (0, 1):
                c = i * 2 + x
                y = 1 - x

                @pl.when(c > 0)
                def _():
                    drain_wb(c - 1, y)

                @pl.when(c + 1 < n_chunks)
                def _():
                    fire_gathers(y)

                drain_gathers(x)
                fire_wb(c, x)

                @pl.when(c + 2 < n_chunks)
                def _():
                    load_idx(c + 2, x)
            return carry

        lax.fori_loop(0, n_chunks // 2, body, 0)
        drain_wb(n_chunks - 1, (n_chunks - 1) % 2)

    return emb


def kernel(residue_type, weight):
    b, h = residue_type.shape
    n = b * h
    idx = residue_type.reshape(n // IPG, IPG).astype(jnp.int32)
    out = _make_emb(n)(idx, weight)
    return out.reshape(b, h, weight.shape[1])
